# initial kernel scaffold (unmeasured)
import jax
import jax.numpy as jnp
from jax import lax
from jax.experimental import pallas as pl
from jax.experimental.pallas import tpu as pltpu

N_DEV = 16
SQ = 1024
SKV = 1024
H_LOCAL = 8
DH = 128
D_MODEL = 1024
CHUNK = SQ // N_DEV
SCALE = 0.08838834764831843


def kernel(x, Wq, K_ext, V_ext, Wo):
    Wq3 = Wq.reshape(D_MODEL, N_DEV, D_MODEL)
    Wo3 = Wo.reshape(N_DEV, D_MODEL, D_MODEL)
    x2 = x.reshape(SQ, D_MODEL)

    def body(x_ref, wq_ref, k_ref, v_ref, wo_ref, out_ref,
             wq_s, wo_s, ctx_s, partial_s, send_buf, recv_buf,
             send_sems, recv_sems, load_sems):
        my = lax.axis_index("i")
        left = lax.rem(my + N_DEV - 1, N_DEV)
        right = lax.rem(my + 1, N_DEV)

        cp_wq = pltpu.make_async_copy(wq_ref.at[:, my, :], wq_s, load_sems.at[0])
        cp_wo = pltpu.make_async_copy(wo_ref.at[my], wo_s, load_sems.at[1])
        cp_wq.start()
        cp_wo.start()

        barrier_sem = pltpu.get_barrier_semaphore()
        for nbr in (left, right):
            pl.semaphore_signal(
                barrier_sem, inc=1,
                device_id=(nbr,), device_id_type=pl.DeviceIdType.MESH,
            )
        pl.semaphore_wait(barrier_sem, 2)

        cp_wq.wait()
        xb = x_ref[...].astype(jnp.bfloat16)
        wqb = wq_s[...].astype(jnp.bfloat16)
        q_all = jnp.dot(xb, wqb, preferred_element_type=jnp.float32)

        qb = lax.broadcasted_iota(jnp.int32, (SQ, SKV), 0) // 64
        kb = lax.broadcasted_iota(jnp.int32, (SQ, SKV), 1) // 64
        mask = (qb == kb) | (kb == 0) | (lax.rem(qb + kb, 3) == 0)
        bias = jnp.where(mask, 0.0, -1e9).astype(jnp.float32)

        for h in range(H_LOCAL):
            q_h = q_all[:, h * DH:(h + 1) * DH].astype(jnp.bfloat16)
            k_h = k_ref[0, :, h, :].astype(jnp.bfloat16)
            v_h = v_ref[0, :, h, :].astype(jnp.bfloat16)
            scores = lax.dot_general(
                q_h, k_h, (((1,), (1,)), ((), ())),
                preferred_element_type=jnp.float32,
            ) * SCALE + bias
            m = jnp.max(scores, axis=1, keepdims=True)
            e = jnp.exp(scores - m)
            w = e / jnp.sum(e, axis=1, keepdims=True)
            ctx_h = jnp.dot(w.astype(jnp.bfloat16), v_h,
                            preferred_element_type=jnp.float32)
            ctx_s[:, h * DH:(h + 1) * DH] = ctx_h.astype(jnp.bfloat16)

        cp_wo.wait()
        wob = wo_s[...].astype(jnp.bfloat16)
        partial_s[...] = jnp.dot(ctx_s[...], wob,
                                 preferred_element_type=jnp.float32)

        def hop(slot, next_val_fn):
            rdma = pltpu.make_async_remote_copy(
                src_ref=send_buf.at[slot],
                dst_ref=recv_buf.at[slot],
                send_sem=send_sems.at[slot],
                recv_sem=recv_sems.at[slot],
                device_id=(right,),
                device_id_type=pl.DeviceIdType.MESH,
            )
            rdma.start()
            rdma.wait()

        def rows(c):
            return pl.ds(c * CHUNK, CHUNK)

        send_buf[0] = partial_s[rows(my), :]
        for s in range(N_DEV - 1):
            slot = s % 2
            hop(slot, None)
            c = lax.rem(my + (2 * N_DEV - 1 - s), N_DEV)
            val = recv_buf[slot] + partial_s[rows(c), :]
            if s < N_DEV - 2:
                send_buf[(s + 1) % 2] = val
            else:
                cown = lax.rem(my + 1, N_DEV)
                out_ref[0, rows(cown), :] = val
                send_buf[(s + 1) % 2] = val

        for g in range(N_DEV - 1):
            slot = (N_DEV - 1 + g) % 2
            hop(slot, None)
            c = lax.rem(my + (N_DEV - g), N_DEV)
            out_ref[0, rows(c), :] = recv_buf[slot]
            if g < N_DEV - 2:
                send_buf[(g + N_DEV) % 2] = recv_buf[slot]

    return pl.pallas_call(
        body,
        out_shape=jax.ShapeDtypeStruct((1, SQ, D_MODEL), jnp.float32),
        in_specs=[
            pl.BlockSpec(memory_space=pltpu.VMEM),
            pl.BlockSpec(memory_space=pltpu.ANY),
            pl.BlockSpec(memory_space=pltpu.VMEM),
            pl.BlockSpec(memory_space=pltpu.VMEM),
            pl.BlockSpec(memory_space=pltpu.ANY),
        ],
        out_specs=pl.BlockSpec(memory_space=pltpu.VMEM),
        scratch_shapes=[
            pltpu.VMEM((D_MODEL, D_MODEL), jnp.float32),
            pltpu.VMEM((D_MODEL, D_MODEL), jnp.float32),
            pltpu.VMEM((SQ, D_MODEL), jnp.bfloat16),
            pltpu.VMEM((SQ, D_MODEL), jnp.float32),
            pltpu.VMEM((2, CHUNK, D_MODEL), jnp.float32),
            pltpu.VMEM((2, CHUNK, D_MODEL), jnp.float32),
            pltpu.SemaphoreType.DMA((2,)),
            pltpu.SemaphoreType.DMA((2,)),
            pltpu.SemaphoreType.DMA((2,)),
        ],
        compiler_params=pltpu.CompilerParams(collective_id=0),
    )(x2, Wq3, K_ext, V_ext, Wo3)


# baseline (device time: 868115 ns/iter reference)
import jax
import jax.numpy as jnp
from jax import lax
from jax.experimental import pallas as pl
from jax.experimental.pallas import tpu as pltpu

N_DEV = 16
SQ = 1024
SKV = 1024
H_LOCAL = 8
DH = 128
D_MODEL = 1024
CHUNK = SQ // N_DEV
SCALE = 0.08838834764831843


def kernel(x, Wq, K_ext, V_ext, Wo):
    Wq3 = Wq.reshape(D_MODEL, N_DEV, D_MODEL)
    Wo3 = Wo.reshape(N_DEV, D_MODEL, D_MODEL)
    x2 = x.reshape(SQ, D_MODEL)

    def body(x_ref, wq_ref, k_ref, v_ref, wo_ref, out_ref,
             wq_s, wo_s, ctx_s, partial_s, send_buf, recv_buf,
             send_sems, recv_sems, load_sems):
        my = lax.axis_index("i")
        left = lax.rem(my + N_DEV - 1, N_DEV)
        right = lax.rem(my + 1, N_DEV)

        cp_wq = pltpu.make_async_copy(wq_ref.at[:, my, :], wq_s, load_sems.at[0])
        cp_wo = pltpu.make_async_copy(wo_ref.at[my], wo_s, load_sems.at[1])
        cp_wq.start()
        cp_wo.start()

        barrier_sem = pltpu.get_barrier_semaphore()
        for nbr in (left, right):
            pl.semaphore_signal(
                barrier_sem, inc=1,
                device_id=(nbr,), device_id_type=pl.DeviceIdType.MESH,
            )
        pl.semaphore_wait(barrier_sem, 2)

        cp_wq.wait()
        xb = x_ref[...].astype(jnp.bfloat16)
        wqb = wq_s[...].astype(jnp.bfloat16)
        q_all = jnp.dot(xb, wqb, preferred_element_type=jnp.float32)

        qb = lax.broadcasted_iota(jnp.int32, (SQ, SKV), 0) // 64
        kb = lax.broadcasted_iota(jnp.int32, (SQ, SKV), 1) // 64
        mask = (qb == kb) | (kb == 0) | (lax.rem(qb + kb, 3) == 0)
        bias = jnp.where(mask, 0.0, -1e9).astype(jnp.float32)

        for h in range(H_LOCAL):
            q_h = q_all[:, h * DH:(h + 1) * DH].astype(jnp.bfloat16)
            k_h = k_ref[0, :, h, :].astype(jnp.bfloat16)
            v_h = v_ref[0, :, h, :].astype(jnp.bfloat16)
            scores = lax.dot_general(
                q_h, k_h, (((1,), (1,)), ((), ())),
                preferred_element_type=jnp.float32,
            ) * SCALE + bias
            m = jnp.max(scores, axis=1, keepdims=True)
            e = jnp.exp(scores - m)
            w = e / jnp.sum(e, axis=1, keepdims=True)
            ctx_h = jnp.dot(w.astype(jnp.bfloat16), v_h,
                            preferred_element_type=jnp.float32)
            ctx_s[:, h * DH:(h + 1) * DH] = ctx_h.astype(jnp.bfloat16)

        cp_wo.wait()
        wob = wo_s[...].astype(jnp.bfloat16)
        partial_s[...] = jnp.dot(ctx_s[...], wob,
                                 preferred_element_type=jnp.float32)

        def hop(slot, next_val_fn):
            rdma = pltpu.make_async_remote_copy(
                src_ref=send_buf.at[slot],
                dst_ref=recv_buf.at[slot],
                send_sem=send_sems.at[slot],
                recv_sem=recv_sems.at[slot],
                device_id=(right,),
                device_id_type=pl.DeviceIdType.MESH,
            )
            rdma.start()
            rdma.wait()

        def rows(c):
            return pl.ds(c * CHUNK, CHUNK)

        send_buf[0] = partial_s[rows(my), :]
        for s in range(N_DEV - 1):
            slot = s % 2
            hop(slot, None)
            c = lax.rem(my + (2 * N_DEV - 1 - s), N_DEV)
            val = recv_buf[slot] + partial_s[rows(c), :]
            if s < N_DEV - 2:
                send_buf[(s + 1) % 2] = val
            else:
                cown = lax.rem(my + 1, N_DEV)
                out_ref[0, rows(cown), :] = val
                send_buf[(s + 1) % 2] = val

        for g in range(N_DEV - 1):
            slot = (N_DEV - 1 + g) % 2
            hop(slot, None)
            c = lax.rem(my + (N_DEV - g), N_DEV)
            out_ref[0, rows(c), :] = recv_buf[slot]
            if g < N_DEV - 2:
                send_buf[(g + N_DEV) % 2] = recv_buf[slot]

    return pl.pallas_call(
        body,
        out_shape=jax.ShapeDtypeStruct((1, SQ, D_MODEL), jnp.float32),
        in_specs=[
            pl.BlockSpec(memory_space=pltpu.VMEM),
            pl.BlockSpec(memory_space=pl.ANY),
            pl.BlockSpec(memory_space=pltpu.VMEM),
            pl.BlockSpec(memory_space=pltpu.VMEM),
            pl.BlockSpec(memory_space=pl.ANY),
        ],
        out_specs=pl.BlockSpec(memory_space=pltpu.VMEM),
        scratch_shapes=[
            pltpu.VMEM((D_MODEL, D_MODEL), jnp.float32),
            pltpu.VMEM((D_MODEL, D_MODEL), jnp.float32),
            pltpu.VMEM((SQ, D_MODEL), jnp.bfloat16),
            pltpu.VMEM((SQ, D_MODEL), jnp.float32),
            pltpu.VMEM((2, CHUNK, D_MODEL), jnp.float32),
            pltpu.VMEM((2, CHUNK, D_MODEL), jnp.float32),
            pltpu.SemaphoreType.DMA((2,)),
            pltpu.SemaphoreType.DMA((2,)),
            pltpu.SemaphoreType.DMA((2,)),
        ],
        compiler_params=pltpu.CompilerParams(collective_id=0),
    )(x2, Wq3, K_ext, V_ext, Wo3)


# device time: 180693 ns/iter; 4.8044x vs baseline; 4.8044x over previous
import jax
import jax.numpy as jnp
from jax import lax
from jax.experimental import pallas as pl
from jax.experimental.pallas import tpu as pltpu

N_DEV = 16
SQ = 1024
SKV = 1024
H_LOCAL = 8
DH = 128
D_MODEL = 1024
CHUNK = SQ // N_DEV
SCALE = 0.08838834764831843


def kernel(x, Wq, K_ext, V_ext, Wo):
    def body(x_ref, wq_ref, k_ref, v_ref, wo_ref, out_ref,
             wq_s, wo_s, ctx_s, partial_s, send_buf, recv_buf,
             send_sems, recv_sems, load_sems):
        my = lax.axis_index("i")
        left = lax.rem(my + N_DEV - 1, N_DEV)
        right = lax.rem(my + 1, N_DEV)

        cp_wq = pltpu.make_async_copy(
            wq_ref.at[:, pl.ds(my * D_MODEL, D_MODEL)], wq_s, load_sems.at[0])
        cp_wo = pltpu.make_async_copy(
            wo_ref.at[pl.ds(my * D_MODEL, D_MODEL), :], wo_s, load_sems.at[1])
        cp_wq.start()
        cp_wo.start()

        barrier_sem = pltpu.get_barrier_semaphore()
        for nbr in (left, right):
            pl.semaphore_signal(
                barrier_sem, inc=1,
                device_id=(nbr,), device_id_type=pl.DeviceIdType.MESH,
            )
        pl.semaphore_wait(barrier_sem, 2)

        cp_wq.wait()
        xb = x_ref[0].astype(jnp.bfloat16)
        wqb = wq_s[...].astype(jnp.bfloat16)
        q_all = jnp.dot(xb, wqb, preferred_element_type=jnp.float32)

        qb = lax.broadcasted_iota(jnp.int32, (SQ, SKV), 0) // 64
        kb = lax.broadcasted_iota(jnp.int32, (SQ, SKV), 1) // 64
        mask = (qb == kb) | (kb == 0) | (lax.rem(qb + kb, 3) == 0)
        bias = jnp.where(mask, 0.0, -1e9).astype(jnp.float32)

        for h in range(H_LOCAL):
            q_h = q_all[:, h * DH:(h + 1) * DH].astype(jnp.bfloat16)
            k_h = k_ref[0, :, h, :].astype(jnp.bfloat16)
            v_h = v_ref[0, :, h, :].astype(jnp.bfloat16)
            scores = lax.dot_general(
                q_h, k_h, (((1,), (1,)), ((), ())),
                preferred_element_type=jnp.float32,
            ) * SCALE + bias
            m = jnp.max(scores, axis=1, keepdims=True)
            e = jnp.exp(scores - m)
            w = e / jnp.sum(e, axis=1, keepdims=True)
            ctx_h = jnp.dot(w.astype(jnp.bfloat16), v_h,
                            preferred_element_type=jnp.float32)
            ctx_s[:, h * DH:(h + 1) * DH] = ctx_h.astype(jnp.bfloat16)

        cp_wo.wait()
        wob = wo_s[...].astype(jnp.bfloat16)
        partial_s[...] = jnp.dot(ctx_s[...], wob,
                                 preferred_element_type=jnp.float32)

        def hop(slot, next_val_fn):
            rdma = pltpu.make_async_remote_copy(
                src_ref=send_buf.at[slot],
                dst_ref=recv_buf.at[slot],
                send_sem=send_sems.at[slot],
                recv_sem=recv_sems.at[slot],
                device_id=(right,),
                device_id_type=pl.DeviceIdType.MESH,
            )
            rdma.start()
            rdma.wait()

        def rows(c):
            return pl.ds(c * CHUNK, CHUNK)

        send_buf[0] = partial_s[rows(my), :]
        for s in range(N_DEV - 1):
            slot = s % 2
            hop(slot, None)
            c = lax.rem(my + (2 * N_DEV - 1 - s), N_DEV)
            val = recv_buf[slot] + partial_s[rows(c), :]
            if s < N_DEV - 2:
                send_buf[(s + 1) % 2] = val
            else:
                cown = lax.rem(my + 1, N_DEV)
                out_ref[0, rows(cown), :] = val
                send_buf[(s + 1) % 2] = val

        for g in range(N_DEV - 1):
            slot = (N_DEV - 1 + g) % 2
            hop(slot, None)
            c = lax.rem(my + (N_DEV - g), N_DEV)
            out_ref[0, rows(c), :] = recv_buf[slot]
            if g < N_DEV - 2:
                send_buf[(g + N_DEV) % 2] = recv_buf[slot]

    return pl.pallas_call(
        body,
        out_shape=jax.ShapeDtypeStruct((1, SQ, D_MODEL), jnp.float32),
        in_specs=[
            pl.BlockSpec(memory_space=pltpu.VMEM),
            pl.BlockSpec(memory_space=pl.ANY),
            pl.BlockSpec(memory_space=pltpu.VMEM),
            pl.BlockSpec(memory_space=pltpu.VMEM),
            pl.BlockSpec(memory_space=pl.ANY),
        ],
        out_specs=pl.BlockSpec(memory_space=pltpu.VMEM),
        scratch_shapes=[
            pltpu.VMEM((D_MODEL, D_MODEL), jnp.float32),
            pltpu.VMEM((D_MODEL, D_MODEL), jnp.float32),
            pltpu.VMEM((SQ, D_MODEL), jnp.bfloat16),
            pltpu.VMEM((SQ, D_MODEL), jnp.float32),
            pltpu.VMEM((2, CHUNK, D_MODEL), jnp.float32),
            pltpu.VMEM((2, CHUNK, D_MODEL), jnp.float32),
            pltpu.SemaphoreType.DMA((2,)),
            pltpu.SemaphoreType.DMA((2,)),
            pltpu.SemaphoreType.DMA((2,)),
        ],
        compiler_params=pltpu.CompilerParams(collective_id=0),
    )(x, Wq, K_ext, V_ext, Wo)


# device time: 176487 ns/iter; 4.9189x vs baseline; 1.0238x over previous
import jax
import jax.numpy as jnp
from jax import lax
from jax.experimental import pallas as pl
from jax.experimental.pallas import tpu as pltpu

N_DEV = 16
SQ = 1024
SKV = 1024
H_LOCAL = 8
DH = 128
D_MODEL = 1024
CHUNK = SQ // N_DEV
SCALE = 0.08838834764831843


def kernel(x, Wq, K_ext, V_ext, Wo):
    def body(x_ref, wq_ref, k_ref, v_ref, wo_ref, out_ref,
             wq_s, wo_s, ctx_s, partial_s,
             send_cw, recv_cw, send_ccw, recv_ccw,
             ssem_cw, rsem_cw, ssem_ccw, rsem_ccw, load_sems):
        my = lax.axis_index("i")
        left = lax.rem(my + N_DEV - 1, N_DEV)
        right = lax.rem(my + 1, N_DEV)

        cp_wq = pltpu.make_async_copy(
            wq_ref.at[:, pl.ds(my * D_MODEL, D_MODEL)], wq_s, load_sems.at[0])
        cp_wo = pltpu.make_async_copy(
            wo_ref.at[pl.ds(my * D_MODEL, D_MODEL), :], wo_s, load_sems.at[1])
        cp_wq.start()
        cp_wo.start()

        barrier_sem = pltpu.get_barrier_semaphore()
        for nbr in (left, right):
            pl.semaphore_signal(
                barrier_sem, inc=1,
                device_id=(nbr,), device_id_type=pl.DeviceIdType.MESH,
            )
        pl.semaphore_wait(barrier_sem, 2)

        cp_wq.wait()
        xb = x_ref[0].astype(jnp.bfloat16)
        wqb = wq_s[...].astype(jnp.bfloat16)
        q_all = jnp.dot(xb, wqb, preferred_element_type=jnp.float32)

        qb = lax.broadcasted_iota(jnp.int32, (SQ, SKV), 0) // 64
        kb = lax.broadcasted_iota(jnp.int32, (SQ, SKV), 1) // 64
        mask = (qb == kb) | (kb == 0) | (lax.rem(qb + kb, 3) == 0)
        bias = jnp.where(mask, 0.0, -1e9).astype(jnp.float32)

        for h in range(H_LOCAL):
            q_h = q_all[:, h * DH:(h + 1) * DH].astype(jnp.bfloat16)
            k_h = k_ref[0, :, h, :].astype(jnp.bfloat16)
            v_h = v_ref[0, :, h, :].astype(jnp.bfloat16)
            scores = lax.dot_general(
                q_h, k_h, (((1,), (1,)), ((), ())),
                preferred_element_type=jnp.float32,
            ) * SCALE + bias
            m = jnp.max(scores, axis=1, keepdims=True)
            e = jnp.exp(scores - m)
            w = e / jnp.sum(e, axis=1, keepdims=True)
            ctx_h = jnp.dot(w.astype(jnp.bfloat16), v_h,
                            preferred_element_type=jnp.float32)
            ctx_s[:, h * DH:(h + 1) * DH] = ctx_h.astype(jnp.bfloat16)

        cp_wo.wait()
        wob = wo_s[...].astype(jnp.bfloat16)
        partial_s[...] = jnp.dot(ctx_s[...], wob,
                                 preferred_element_type=jnp.float32)

        HALF = D_MODEL // 2

        def hop(slot):
            r_cw = pltpu.make_async_remote_copy(
                src_ref=send_cw.at[slot], dst_ref=recv_cw.at[slot],
                send_sem=ssem_cw.at[slot], recv_sem=rsem_cw.at[slot],
                device_id=(right,), device_id_type=pl.DeviceIdType.MESH,
            )
            r_ccw = pltpu.make_async_remote_copy(
                src_ref=send_ccw.at[slot], dst_ref=recv_ccw.at[slot],
                send_sem=ssem_ccw.at[slot], recv_sem=rsem_ccw.at[slot],
                device_id=(left,), device_id_type=pl.DeviceIdType.MESH,
            )
            r_cw.start()
            r_ccw.start()
            r_cw.wait()
            r_ccw.wait()

        def rows(c):
            return pl.ds(c * CHUNK, CHUNK)

        send_cw[0] = partial_s[rows(my), :HALF]
        send_ccw[0] = partial_s[rows(my), HALF:]
        for s in range(N_DEV - 1):
            slot = s % 2
            hop(slot)
            c_cw = lax.rem(my + (2 * N_DEV - 1 - s), N_DEV)
            c_ccw = lax.rem(my + 1 + s, N_DEV)
            v_cw = recv_cw[slot] + partial_s[rows(c_cw), :HALF]
            v_ccw = recv_ccw[slot] + partial_s[rows(c_ccw), HALF:]
            if s < N_DEV - 2:
                send_cw[(s + 1) % 2] = v_cw
                send_ccw[(s + 1) % 2] = v_ccw
            else:
                out_ref[0, rows(lax.rem(my + 1, N_DEV)), :HALF] = v_cw
                out_ref[0, rows(lax.rem(my + N_DEV - 1, N_DEV)), HALF:] = v_ccw
                send_cw[(s + 1) % 2] = v_cw
                send_ccw[(s + 1) % 2] = v_ccw

        for g in range(N_DEV - 1):
            slot = (N_DEV - 1 + g) % 2
            hop(slot)
            c_cw = lax.rem(my + (N_DEV - g), N_DEV)
            c_ccw = lax.rem(my + g, N_DEV)
            out_ref[0, rows(c_cw), :HALF] = recv_cw[slot]
            out_ref[0, rows(c_ccw), HALF:] = recv_ccw[slot]
            if g < N_DEV - 2:
                send_cw[(g + N_DEV) % 2] = recv_cw[slot]
                send_ccw[(g + N_DEV) % 2] = recv_ccw[slot]

    return pl.pallas_call(
        body,
        out_shape=jax.ShapeDtypeStruct((1, SQ, D_MODEL), jnp.float32),
        in_specs=[
            pl.BlockSpec(memory_space=pltpu.VMEM),
            pl.BlockSpec(memory_space=pl.ANY),
            pl.BlockSpec(memory_space=pltpu.VMEM),
            pl.BlockSpec(memory_space=pltpu.VMEM),
            pl.BlockSpec(memory_space=pl.ANY),
        ],
        out_specs=pl.BlockSpec(memory_space=pltpu.VMEM),
        scratch_shapes=[
            pltpu.VMEM((D_MODEL, D_MODEL), jnp.float32),
            pltpu.VMEM((D_MODEL, D_MODEL), jnp.float32),
            pltpu.VMEM((SQ, D_MODEL), jnp.bfloat16),
            pltpu.VMEM((SQ, D_MODEL), jnp.float32),
            pltpu.VMEM((2, CHUNK, D_MODEL // 2), jnp.float32),
            pltpu.VMEM((2, CHUNK, D_MODEL // 2), jnp.float32),
            pltpu.VMEM((2, CHUNK, D_MODEL // 2), jnp.float32),
            pltpu.VMEM((2, CHUNK, D_MODEL // 2), jnp.float32),
            pltpu.SemaphoreType.DMA((2,)),
            pltpu.SemaphoreType.DMA((2,)),
            pltpu.SemaphoreType.DMA((2,)),
            pltpu.SemaphoreType.DMA((2,)),
            pltpu.SemaphoreType.DMA((2,)),
        ],
        compiler_params=pltpu.CompilerParams(collective_id=0),
    )(x, Wq, K_ext, V_ext, Wo)
